# Initial kernel scaffold; baseline (speedup 1.0000x reference)
import functools

import jax
import jax.numpy as jnp
from jax import lax
from jax.experimental import pallas as pl
from jax.experimental.pallas import tpu as pltpu
from jax.experimental.pallas import tpu_sc as plsc

NS = 4
K = 1024
SD = 64
D = NS * SD
B, N = 16, 1024
ROWS = B * N
RB = 512
NB = ROWS // RB
DECAY = 0.99
LOSS_SCALE = 1.25 / (NS * ROWS * SD)


def _tc_body(x_ref, cb_ref, cc_ref, nn_ref, nnb_ref, counts_ref, loss_ref,
             acc_ref):
    b = pl.program_id(0)
    step_sum = jnp.float32(0.0)
    for s in range(NS):
        xs = x_ref[:, s * SD:(s + 1) * SD]
        cs = cb_ref[s]
        x2 = jnp.sum(xs * xs, axis=1, keepdims=True)
        c2 = jnp.sum(cs * cs, axis=1)[None, :]
        xc = lax.dot_general(xs, cs, (((1,), (1,)), ((), ())),
                             preferred_element_type=jnp.float32)
        d = (x2 - 2.0 * xc) + c2
        m = jnp.min(d, axis=1, keepdims=True)
        iota = lax.broadcasted_iota(jnp.int32, (RB, K), 1)
        idx = jnp.min(jnp.where(d == m, iota, K), axis=1)
        nn_ref[s, 0, 0, :] = idx
        nnb_ref[s, 0, 0, :] = idx + s * K
        cnt = jnp.sum((idx[:, None] == iota).astype(jnp.float32), axis=0)
        step_sum = step_sum + jnp.sum(m)

        @pl.when(b == 0)
        def _():
            counts_ref[s, 0, :] = cnt

        @pl.when(b > 0)
        def _():
            counts_ref[s, 0, :] = counts_ref[s, 0, :] + cnt

        @pl.when(b == NB - 1)
        def _():
            counts_ref[s, 0, :] = (DECAY * cc_ref[s, 0, :]
                                   + (1.0 - DECAY) * counts_ref[s, 0, :])

    @pl.when(b == 0)
    def _():
        acc_ref[0] = step_sum

    @pl.when(b > 0)
    def _():
        acc_ref[0] = acc_ref[0] + step_sum

    @pl.when(b == NB - 1)
    def _():
        loss_ref[0, 0] = acc_ref[0] * LOSS_SCALE


def _tc_call(x2d, codebooks, cc3):
    return pl.pallas_call(
        _tc_body,
        grid=(NB,),
        in_specs=[
            pl.BlockSpec((RB, D), lambda b: (b, 0)),
            pl.BlockSpec((NS, K, SD), lambda b: (0, 0, 0)),
            pl.BlockSpec((NS, 1, K), lambda b: (0, 0, 0)),
        ],
        out_specs=[
            pl.BlockSpec((NS, 1, 1, RB), lambda b: (0, b, 0, 0)),
            pl.BlockSpec((NS, 1, 1, RB), lambda b: (0, b, 0, 0)),
            pl.BlockSpec((NS, 1, K), lambda b: (0, 0, 0)),
            pl.BlockSpec(memory_space=pltpu.SMEM),
        ],
        out_shape=[
            jax.ShapeDtypeStruct((NS, NB, 1, RB), jnp.int32),
            jax.ShapeDtypeStruct((NS, NB, 1, RB), jnp.int32),
            jax.ShapeDtypeStruct((NS, 1, K), jnp.float32),
            jax.ShapeDtypeStruct((1, 1), jnp.float32),
        ],
        scratch_shapes=[pltpu.SMEM((1,), jnp.float32)],
        compiler_params=pltpu.CompilerParams(
            dimension_semantics=("arbitrary",)),
    )(x2d, codebooks, cc3)


CH = 128
NW = 32
CPW = NS * ROWS // NW // CH


def _sc_gather(idx2d, cbflat):
    mesh = plsc.VectorSubcoreMesh(core_axis_name="c", subcore_axis_name="s")

    @functools.partial(
        pl.kernel,
        out_type=jax.ShapeDtypeStruct((ROWS, D), jnp.float32),
        mesh=mesh,
        scratch_types=[
            pltpu.VMEM((CPW, CH), jnp.int32),
            pltpu.VMEM((CH, SD), jnp.float32),
            pltpu.VMEM((CH, SD), jnp.float32),
            pltpu.SemaphoreType.DMA,
            pltpu.SemaphoreType.DMA,
        ],
    )
    def k(idx_hbm, cb_hbm, out_hbm, idx_v, buf0, buf1, sem0, sem1):
        wid = lax.axis_index("s") * 2 + lax.axis_index("c")
        pltpu.sync_copy(idx_hbm.at[pl.ds(wid * CPW, CPW)], idx_v)
        sec = (wid * CPW) // (ROWS // CH)
        n0 = wid * CPW * CH - sec * ROWS
        bufs = (buf0, buf1)
        sems = (sem0, sem1)
        pending = [pltpu.async_copy(cb_hbm.at[idx_v.at[0]], buf0, sem0)]
        for c in range(CPW):
            cur = bufs[c % 2]
            pending[c].wait()
            if c + 1 < CPW:
                pending.append(pltpu.async_copy(
                    cb_hbm.at[idx_v.at[c + 1]], bufs[(c + 1) % 2],
                    sems[(c + 1) % 2]))
            pltpu.sync_copy(
                cur, out_hbm.at[pl.ds(n0 + c * CH, CH), pl.ds(sec * SD, SD)])

    return k(idx2d, cbflat)


def kernel(inputs, train, codebooks, cluster_counts):
    x2d = inputs.reshape(ROWS, D)
    nn4, nnb4, counts_ema, loss11 = _tc_call(
        x2d, codebooks, cluster_counts.reshape(NS, 1, K))
    q2d = _sc_gather(nnb4.reshape(NS * ROWS // CH, CH),
                     codebooks.reshape(NS * K, SD))
    quantized = q2d.reshape(B, N, D)
    nn_idx = nn4.reshape(NS, B, N)
    counts = jnp.where(train, counts_ema.reshape(NS, K), cluster_counts)
    loss = loss11.reshape(())
    codebook = lax.stop_gradient(codebooks.reshape(NS * K, SD))
    return quantized, loss, nn_idx, codebook, counts


# speedup vs baseline: 1.3248x; 1.3248x over previous
import functools

import jax
import jax.numpy as jnp
from jax import lax
from jax.experimental import pallas as pl
from jax.experimental.pallas import tpu as pltpu
from jax.experimental.pallas import tpu_sc as plsc

NS = 4
K = 1024
SD = 64
D = NS * SD
B, N = 16, 1024
ROWS = B * N
RB = 512
NB = ROWS // RB
DECAY = 0.99
LOSS_SCALE = 1.25 / (NS * ROWS * SD)


def _tc_body(x_ref, cb_ref, cc_ref, nn_ref, nnb_ref, counts_ref, loss_ref,
             acc_ref):
    b = pl.program_id(0)
    step_sum = jnp.float32(0.0)
    for s in range(NS):
        xs = x_ref[:, s * SD:(s + 1) * SD]
        cs = cb_ref[s]
        x2 = jnp.sum(xs * xs, axis=1, keepdims=True)
        c2 = jnp.sum(cs * cs, axis=1)[None, :]
        xc = lax.dot_general(xs, cs, (((1,), (1,)), ((), ())),
                             preferred_element_type=jnp.float32)
        d = (x2 - 2.0 * xc) + c2
        m = jnp.min(d, axis=1, keepdims=True)
        iota = lax.broadcasted_iota(jnp.int32, (RB, K), 1)
        idx = jnp.min(jnp.where(d == m, iota, K), axis=1)
        nn_ref[s, 0, 0, :] = idx
        nnb_ref[s, 0, 0, :] = idx + s * K
        cnt = jnp.sum((idx[:, None] == iota).astype(jnp.float32), axis=0)
        step_sum = step_sum + jnp.sum(m)

        @pl.when(b == 0)
        def _():
            counts_ref[s, 0, :] = cnt

        @pl.when(b > 0)
        def _():
            counts_ref[s, 0, :] = counts_ref[s, 0, :] + cnt

        @pl.when(b == NB - 1)
        def _():
            counts_ref[s, 0, :] = (DECAY * cc_ref[s, 0, :]
                                   + (1.0 - DECAY) * counts_ref[s, 0, :])

    @pl.when(b == 0)
    def _():
        acc_ref[0] = step_sum

    @pl.when(b > 0)
    def _():
        acc_ref[0] = acc_ref[0] + step_sum

    @pl.when(b == NB - 1)
    def _():
        loss_ref[0, 0] = acc_ref[0] * LOSS_SCALE


def _tc_call(x2d, codebooks, cc3):
    return pl.pallas_call(
        _tc_body,
        grid=(NB,),
        in_specs=[
            pl.BlockSpec((RB, D), lambda b: (b, 0)),
            pl.BlockSpec((NS, K, SD), lambda b: (0, 0, 0)),
            pl.BlockSpec((NS, 1, K), lambda b: (0, 0, 0)),
        ],
        out_specs=[
            pl.BlockSpec((NS, 1, 1, RB), lambda b: (0, b, 0, 0)),
            pl.BlockSpec((NS, 1, 1, RB), lambda b: (0, b, 0, 0)),
            pl.BlockSpec((NS, 1, K), lambda b: (0, 0, 0)),
            pl.BlockSpec(memory_space=pltpu.SMEM),
        ],
        out_shape=[
            jax.ShapeDtypeStruct((NS, NB, 1, RB), jnp.int32),
            jax.ShapeDtypeStruct((NS, NB, 1, RB), jnp.int32),
            jax.ShapeDtypeStruct((NS, 1, K), jnp.float32),
            jax.ShapeDtypeStruct((1, 1), jnp.float32),
        ],
        scratch_shapes=[pltpu.SMEM((1,), jnp.float32)],
        compiler_params=pltpu.CompilerParams(
            dimension_semantics=("arbitrary",)),
    )(x2d, codebooks, cc3)


NW = 32
CH = 128
CPW = NS * ROWS // NW // CH
PD = 128


def _sc_gather(idx2d, cbpad):
    mesh = plsc.VectorSubcoreMesh(core_axis_name="c", subcore_axis_name="s")

    @functools.partial(
        pl.kernel,
        out_type=jax.ShapeDtypeStruct((NS, ROWS, PD), jnp.float32),
        mesh=mesh,
        scratch_types=[
            pltpu.VMEM((CPW, CH), jnp.int32),
            pltpu.VMEM((CH, PD), jnp.float32),
            pltpu.VMEM((CH, PD), jnp.float32),
            pltpu.SemaphoreType.DMA,
            pltpu.SemaphoreType.DMA,
        ],
    )
    def k(idx_hbm, cb_hbm, out_hbm, idx_v, buf0, buf1, sem0, sem1):
        wid = lax.axis_index("s") * 2 + lax.axis_index("c")
        sec = (wid * CPW) // (ROWS // CH)
        n0 = wid * CPW * CH - sec * ROWS
        pltpu.sync_copy(idx_hbm.at[pl.ds(wid * CPW, CPW)], idx_v)
        bufs = (buf0, buf1)
        sems = (sem0, sem1)
        pending = pltpu.async_copy(cb_hbm.at[idx_v.at[0]], buf0, sem0)
        for c in range(CPW):
            pending.wait()
            if c + 1 < CPW:
                pending = pltpu.async_copy(
                    cb_hbm.at[idx_v.at[c + 1]], bufs[(c + 1) % 2],
                    sems[(c + 1) % 2])
            pltpu.sync_copy(bufs[c % 2],
                            out_hbm.at[sec, pl.ds(n0 + c * CH, CH)])

    return k(idx2d, cbpad)


def kernel(inputs, train, codebooks, cluster_counts):
    x2d = inputs.reshape(ROWS, D)
    nn4, nnb4, counts_ema, loss11 = _tc_call(
        x2d, codebooks, cluster_counts.reshape(NS, 1, K))
    cbpad = jnp.pad(codebooks.reshape(NS * K, SD), ((0, 0), (0, PD - SD)))
    q3 = _sc_gather(nnb4.reshape(NS * ROWS // CH, CH), cbpad)
    quantized = q3[:, :, :SD].transpose(1, 0, 2).reshape(B, N, D)
    nn_idx = nn4.reshape(NS, B, N)
    counts = jnp.where(train, counts_ema.reshape(NS, K), cluster_counts)
    loss = loss11.reshape(())
    codebook = lax.stop_gradient(codebooks.reshape(NS * K, SD))
    return quantized, loss, nn_idx, codebook, counts


# speedup vs baseline: 1.4855x; 1.1213x over previous
import functools

import jax
import jax.numpy as jnp
from jax import lax
from jax.experimental import pallas as pl
from jax.experimental.pallas import tpu as pltpu
from jax.experimental.pallas import tpu_sc as plsc

NS = 4
K = 1024
SD = 64
D = NS * SD
B, N = 16, 1024
ROWS = B * N
RB = 512
NB = ROWS // RB
DECAY = 0.99
LOSS_SCALE = 1.25 / (NS * ROWS * SD)


def _tc_body(x_ref, cb_ref, cc_ref, nn_ref, nnb_ref, counts_ref, loss_ref,
             acc_ref):
    b = pl.program_id(0)
    step_sum = jnp.float32(0.0)
    iota = lax.broadcasted_iota(jnp.int32, (RB, K), 1).astype(jnp.float32)
    for s in range(NS):
        xs = x_ref[:, s * SD:(s + 1) * SD]
        cs = cb_ref[s]
        x2 = jnp.sum(xs * xs, axis=1, keepdims=True)
        c2 = jnp.sum(cs * cs, axis=1)[None, :]
        xc = lax.dot_general(xs, cs, (((1,), (1,)), ((), ())),
                             preferred_element_type=jnp.float32)
        d = (x2 - 2.0 * xc) + c2
        m = jnp.min(d, axis=1, keepdims=True)
        mask = d == m
        idx = jnp.min(jnp.where(mask, iota, jnp.float32(K)),
                      axis=1).astype(jnp.int32)
        nn_ref[s, 0, 0, :] = idx
        nnb_ref[s, 0, 0, :] = idx + s * K
        cnt = jnp.sum(mask.astype(jnp.float32), axis=0)
        step_sum = step_sum + jnp.sum(m)

        @pl.when(b == 0)
        def _():
            counts_ref[s, 0, :] = cnt

        @pl.when(b > 0)
        def _():
            counts_ref[s, 0, :] = counts_ref[s, 0, :] + cnt

        @pl.when(b == NB - 1)
        def _():
            counts_ref[s, 0, :] = (DECAY * cc_ref[s, 0, :]
                                   + (1.0 - DECAY) * counts_ref[s, 0, :])

    @pl.when(b == 0)
    def _():
        acc_ref[0] = step_sum

    @pl.when(b > 0)
    def _():
        acc_ref[0] = acc_ref[0] + step_sum

    @pl.when(b == NB - 1)
    def _():
        loss_ref[0, 0] = acc_ref[0] * LOSS_SCALE


def _tc_call(x2d, codebooks, cc3):
    return pl.pallas_call(
        _tc_body,
        grid=(NB,),
        in_specs=[
            pl.BlockSpec((RB, D), lambda b: (b, 0)),
            pl.BlockSpec((NS, K, SD), lambda b: (0, 0, 0)),
            pl.BlockSpec((NS, 1, K), lambda b: (0, 0, 0)),
        ],
        out_specs=[
            pl.BlockSpec((NS, 1, 1, RB), lambda b: (0, b, 0, 0)),
            pl.BlockSpec((NS, 1, 1, RB), lambda b: (0, b, 0, 0)),
            pl.BlockSpec((NS, 1, K), lambda b: (0, 0, 0)),
            pl.BlockSpec(memory_space=pltpu.SMEM),
        ],
        out_shape=[
            jax.ShapeDtypeStruct((NS, NB, 1, RB), jnp.int32),
            jax.ShapeDtypeStruct((NS, NB, 1, RB), jnp.int32),
            jax.ShapeDtypeStruct((NS, 1, K), jnp.float32),
            jax.ShapeDtypeStruct((1, 1), jnp.float32),
        ],
        scratch_shapes=[pltpu.SMEM((1,), jnp.float32)],
        compiler_params=pltpu.CompilerParams(
            dimension_semantics=("arbitrary",)),
    )(x2d, codebooks, cc3)


NW = 32
CH = 128
CPW = NS * ROWS // NW // CH
PD = 128


def _sc_gather(idx2d, cbpad):
    mesh = plsc.VectorSubcoreMesh(core_axis_name="c", subcore_axis_name="s")

    @functools.partial(
        pl.kernel,
        out_type=jax.ShapeDtypeStruct((NS, ROWS, PD), jnp.float32),
        mesh=mesh,
        scratch_types=[
            pltpu.VMEM((CPW, CH), jnp.int32),
            pltpu.VMEM((CH, PD), jnp.float32),
            pltpu.VMEM((CH, PD), jnp.float32),
            pltpu.SemaphoreType.DMA,
            pltpu.SemaphoreType.DMA,
        ],
    )
    def k(idx_hbm, cb_hbm, out_hbm, idx_v, buf0, buf1, sem0, sem1):
        wid = lax.axis_index("s") * 2 + lax.axis_index("c")
        sec = (wid * CPW) // (ROWS // CH)
        n0 = wid * CPW * CH - sec * ROWS
        pltpu.sync_copy(idx_hbm.at[pl.ds(wid * CPW, CPW)], idx_v)
        bufs = (buf0, buf1)
        sems = (sem0, sem1)
        pending = pltpu.async_copy(cb_hbm.at[idx_v.at[0]], buf0, sem0)
        for c in range(CPW):
            pending.wait()
            if c + 1 < CPW:
                pending = pltpu.async_copy(
                    cb_hbm.at[idx_v.at[c + 1]], bufs[(c + 1) % 2],
                    sems[(c + 1) % 2])
            pltpu.sync_copy(bufs[c % 2],
                            out_hbm.at[sec, pl.ds(n0 + c * CH, CH)])

    return k(idx2d, cbpad)


def kernel(inputs, train, codebooks, cluster_counts):
    x2d = inputs.reshape(ROWS, D)
    nn4, nnb4, counts_ema, loss11 = _tc_call(
        x2d, codebooks, cluster_counts.reshape(NS, 1, K))
    cbpad = jnp.pad(codebooks.reshape(NS * K, SD), ((0, 0), (0, PD - SD)))
    q3 = _sc_gather(nnb4.reshape(NS * ROWS // CH, CH), cbpad)
    quantized = q3[:, :, :SD].transpose(1, 0, 2).reshape(B, N, D)
    nn_idx = nn4.reshape(NS, B, N)
    counts = jnp.where(train, counts_ema.reshape(NS, K), cluster_counts)
    loss = loss11.reshape(())
    codebook = lax.stop_gradient(codebooks.reshape(NS * K, SD))
    return quantized, loss, nn_idx, codebook, counts


# speedup vs baseline: 1.5580x; 1.0488x over previous
import functools

import jax
import jax.numpy as jnp
from jax import lax
from jax.experimental import pallas as pl
from jax.experimental.pallas import tpu as pltpu
from jax.experimental.pallas import tpu_sc as plsc

NS = 4
K = 1024
SD = 64
D = NS * SD
B, N = 16, 1024
ROWS = B * N
RB = 1024
NB = ROWS // RB
DECAY = 0.99
LOSS_SCALE = 1.25 / (NS * ROWS * SD)


def _tc_body(x_ref, cb_ref, cc_ref, nn_ref, nnb_ref, counts_ref, loss_ref,
             acc_ref):
    b = pl.program_id(0)
    step_sum = jnp.float32(0.0)
    iota = lax.broadcasted_iota(jnp.int32, (RB, K), 1).astype(jnp.float32)
    for s in range(NS):
        xs = x_ref[:, s * SD:(s + 1) * SD]
        cs = cb_ref[s]
        x2 = jnp.sum(xs * xs, axis=1, keepdims=True)
        c2 = jnp.sum(cs * cs, axis=1)[None, :]
        xc = lax.dot_general(xs, cs, (((1,), (1,)), ((), ())),
                             preferred_element_type=jnp.float32)
        d = (x2 - 2.0 * xc) + c2
        m = jnp.min(d, axis=1, keepdims=True)
        mask = d == m
        idx = jnp.min(jnp.where(mask, iota, jnp.float32(K)),
                      axis=1).astype(jnp.int32)
        nn_ref[s, 0, 0, :] = idx
        nnb_ref[s, 0, 0, :] = idx + s * K
        cnt = jnp.sum(mask.astype(jnp.float32), axis=0)
        step_sum = step_sum + jnp.sum(m)

        @pl.when(b == 0)
        def _():
            counts_ref[s, 0, :] = cnt

        @pl.when(b > 0)
        def _():
            counts_ref[s, 0, :] = counts_ref[s, 0, :] + cnt

        @pl.when(b == NB - 1)
        def _():
            counts_ref[s, 0, :] = (DECAY * cc_ref[s, 0, :]
                                   + (1.0 - DECAY) * counts_ref[s, 0, :])

    @pl.when(b == 0)
    def _():
        acc_ref[0] = step_sum

    @pl.when(b > 0)
    def _():
        acc_ref[0] = acc_ref[0] + step_sum

    @pl.when(b == NB - 1)
    def _():
        loss_ref[0, 0] = acc_ref[0] * LOSS_SCALE


def _tc_call(x2d, codebooks, cc3):
    return pl.pallas_call(
        _tc_body,
        grid=(NB,),
        in_specs=[
            pl.BlockSpec((RB, D), lambda b: (b, 0)),
            pl.BlockSpec((NS, K, SD), lambda b: (0, 0, 0)),
            pl.BlockSpec((NS, 1, K), lambda b: (0, 0, 0)),
        ],
        out_specs=[
            pl.BlockSpec((NS, 1, 1, RB), lambda b: (0, b, 0, 0)),
            pl.BlockSpec((NS, 1, 1, RB), lambda b: (0, b, 0, 0)),
            pl.BlockSpec((NS, 1, K), lambda b: (0, 0, 0)),
            pl.BlockSpec(memory_space=pltpu.SMEM),
        ],
        out_shape=[
            jax.ShapeDtypeStruct((NS, NB, 1, RB), jnp.int32),
            jax.ShapeDtypeStruct((NS, NB, 1, RB), jnp.int32),
            jax.ShapeDtypeStruct((NS, 1, K), jnp.float32),
            jax.ShapeDtypeStruct((1, 1), jnp.float32),
        ],
        scratch_shapes=[pltpu.SMEM((1,), jnp.float32)],
        compiler_params=pltpu.CompilerParams(
            dimension_semantics=("arbitrary",)),
    )(x2d, codebooks, cc3)


NW = 32
CH = 128
CPW = NS * ROWS // NW // CH
PD = 128


def _sc_gather(idx2d, cbpad):
    mesh = plsc.VectorSubcoreMesh(core_axis_name="c", subcore_axis_name="s")

    @functools.partial(
        pl.kernel,
        out_type=jax.ShapeDtypeStruct((NS, ROWS, PD), jnp.float32),
        mesh=mesh,
        scratch_types=[
            pltpu.VMEM((CPW, CH), jnp.int32),
            pltpu.VMEM((CH, PD), jnp.float32),
            pltpu.VMEM((CH, PD), jnp.float32),
            pltpu.SemaphoreType.DMA,
            pltpu.SemaphoreType.DMA,
        ],
    )
    def k(idx_hbm, cb_hbm, out_hbm, idx_v, buf0, buf1, sem0, sem1):
        wid = lax.axis_index("s") * 2 + lax.axis_index("c")
        sec = (wid * CPW) // (ROWS // CH)
        n0 = wid * CPW * CH - sec * ROWS
        pltpu.sync_copy(idx_hbm.at[pl.ds(wid * CPW, CPW)], idx_v)
        bufs = (buf0, buf1)
        sems = (sem0, sem1)
        pending = pltpu.async_copy(cb_hbm.at[idx_v.at[0]], buf0, sem0)
        for c in range(CPW):
            pending.wait()
            if c + 1 < CPW:
                pending = pltpu.async_copy(
                    cb_hbm.at[idx_v.at[c + 1]], bufs[(c + 1) % 2],
                    sems[(c + 1) % 2])
            pltpu.sync_copy(bufs[c % 2],
                            out_hbm.at[sec, pl.ds(n0 + c * CH, CH)])

    return k(idx2d, cbpad)


def kernel(inputs, train, codebooks, cluster_counts):
    x2d = inputs.reshape(ROWS, D)
    nn4, nnb4, counts_ema, loss11 = _tc_call(
        x2d, codebooks, cluster_counts.reshape(NS, 1, K))
    cbpad = jnp.pad(codebooks.reshape(NS * K, SD), ((0, 0), (0, PD - SD)))
    q3 = _sc_gather(nnb4.reshape(NS * ROWS // CH, CH), cbpad)
    quantized = q3[:, :, :SD].transpose(1, 0, 2).reshape(B, N, D)
    nn_idx = nn4.reshape(NS, B, N)
    counts = jnp.where(train, counts_ema.reshape(NS, K), cluster_counts)
    loss = loss11.reshape(())
    codebook = lax.stop_gradient(codebooks.reshape(NS * K, SD))
    return quantized, loss, nn_idx, codebook, counts
